# baseline (speedup 1.0000x reference)
import functools

import jax
import jax.numpy as jnp
from jax import lax
from jax.experimental import pallas as pl
from jax.experimental.pallas import tpu as pltpu
from jax.experimental.pallas import tpu_sc as plsc

BATCH = 16384
EMBED_K = 32
L = 16
NW = 32
B_PER_W = BATCH // NW
N_IDX_ROWS = B_PER_W // 128


def _body(x_hbm, w_hbm, h_hbm, out_hbm,
          xv, uidx, vidx, rows_u, rows_v, out_v, sem):
    wid = lax.axis_index("s") * 2 + lax.axis_index("c")
    base = wid * B_PER_W

    pltpu.sync_copy(x_hbm.at[pl.ds(base, B_PER_W)], xv)

    lane = lax.iota(jnp.int32, L)
    col0 = jnp.zeros((L,), jnp.int32)
    col1 = jnp.ones((L,), jnp.int32)
    for c in range(B_PER_W // L):
        rows = lane + (c * L)
        u = plsc.load_gather(xv, [rows, col0])
        v = plsc.load_gather(xv, [rows, col1])
        j, off = c // 8, (c % 8) * L
        uidx[j, pl.ds(off, L)] = u
        vidx[j, pl.ds(off, L)] = v

    copies = []
    for j in range(N_IDX_ROWS):
        dst = rows_u.at[pl.ds(j * 128, 128)]
        copies.append(pltpu.async_copy(w_hbm.at[uidx.at[j]], dst, sem))
    for j in range(N_IDX_ROWS):
        dst = rows_v.at[pl.ds(j * 128, 128)]
        copies.append(pltpu.async_copy(h_hbm.at[vidx.at[j]], dst, sem))
    for cp in copies:
        cp.wait()

    def chunk(c, carry):
        rows = lane + c * L
        acc = jnp.zeros((L,), jnp.float32)
        for k in range(EMBED_K):
            colk = jnp.full((L,), k, jnp.int32)
            uk = plsc.load_gather(rows_u, [rows, colk])
            vk = plsc.load_gather(rows_v, [rows, colk])
            acc = acc + uk * vk
        out_v[pl.ds(c * L, L)] = acc
        return carry

    lax.fori_loop(0, B_PER_W // L, chunk, 0)

    pltpu.sync_copy(out_v, out_hbm.at[pl.ds(base, B_PER_W)])


@jax.jit
def kernel(x, W, H):
    mesh = plsc.VectorSubcoreMesh(core_axis_name="c", subcore_axis_name="s")
    f = functools.partial(
        pl.kernel,
        mesh=mesh,
        compiler_params=pltpu.CompilerParams(
            needs_layout_passes=False, use_tc_tiling_on_sc=False),
        out_type=jax.ShapeDtypeStruct((BATCH,), jnp.float32),
        scratch_types=[
            pltpu.VMEM((B_PER_W, 2), jnp.int32),
            pltpu.VMEM((N_IDX_ROWS, 128), jnp.int32),
            pltpu.VMEM((N_IDX_ROWS, 128), jnp.int32),
            pltpu.VMEM((B_PER_W, EMBED_K), jnp.float32),
            pltpu.VMEM((B_PER_W, EMBED_K), jnp.float32),
            pltpu.VMEM((B_PER_W,), jnp.float32),
            pltpu.SemaphoreType.DMA,
        ],
    )(_body)
    return f(x, W, H)
